# baseline (speedup 1.0000x reference)
import functools
import numpy as np
import jax
import jax.numpy as jnp
from jax import lax
from jax.experimental import pallas as pl
from jax.experimental.pallas import tpu as pltpu
from jax.experimental.pallas import tpu_sc as plsc

N = 100000
E = 1600000
EP = 1638400
NA = 102400
PR = NA * 32 // 128
TRASH = N
BR = 256
NBLK = PR // BR
EPS = 1e-5


def _silu(v):
    return v * (1.0 / (1.0 + jnp.exp(-v)))


def _msum(h, i):
    rid = lax.broadcasted_iota(jnp.int32, h.shape, 0)
    lid = lax.broadcasted_iota(jnp.int32, h.shape, 1)
    node = (i * BR + rid) * 4 + lid // 32
    hm = h * (node < N).astype(jnp.float32)
    return jnp.sum(hm), jnp.sum(hm * h)


def _tables_body(emb_ref, w0, w1, w2, w3a, w3b, b0, b1, b2, b3a, b3b,
                 t0, t1, t2, t3a, t3b):
    e = emb_ref[...]
    for w, b, o in ((w0, b0, t0), (w1, b1, t1), (w2, b2, t2),
                    (w3a, b3a, t3a), (w3b, b3b, t3b)):
        o[...] = _silu(jnp.dot(e, w[...], preferred_element_type=jnp.float32)
                       + b[...])


def _make_tables(embp, wts, bs):
    out = tuple(jax.ShapeDtypeStruct((1024, 32), jnp.float32) for _ in range(5))
    return pl.pallas_call(_tables_body, out_shape=out)(embp, *wts, *bs)


def _stats_update(s_ref, p, i, bs, bs2):
    @pl.when(jnp.logical_and(p == 0, i == 0))
    def _():
        s_ref[0] = bs
        s_ref[1] = bs2

    @pl.when(jnp.logical_and(p == 0, i > 0))
    def _():
        s_ref[0] += bs
        s_ref[1] += bs2


def _ln_apply(h, s_ref, count, g_ref, b_ref):
    mu = s_ref[0] / count
    var = s_ref[1] / count - mu * mu
    rs = lax.rsqrt(var + EPS)
    return (h - mu) * rs * g_ref[...] + b_ref[...]


def _layer0_body(x_ref, g0_ref, g_ref, b_ref, wa_ref, wr_ref,
                 hp_ref, rt_ref, s_ref):
    p = pl.program_id(0)
    i = pl.program_id(1)
    h = x_ref[...] + g0_ref[...]
    bs, bs2 = _msum(h, i)
    _stats_update(s_ref, p, i, bs, bs2)

    @pl.when(p == 1)
    def _():
        hn = _ln_apply(h, s_ref, float(N * 32), g_ref, b_ref)
        hp_ref[...] = jnp.dot(hn, wa_ref[...],
                              preferred_element_type=jnp.float32)
        rt_ref[...] = jnp.dot(hn, wr_ref[...],
                              preferred_element_type=jnp.float32)


def _mid_body(emit_d0, agg_ref, dga_ref, dgb_ref, rtp_ref, gte_ref,
              bc_ref, g_ref, b_ref, wa_ref, wr_ref, *rest):
    if emit_d0:
        hp_ref, rt_ref, d0_ref, s_ref = rest
    else:
        hp_ref, rt_ref, s_ref = rest
    p = pl.program_id(0)
    i = pl.program_id(1)
    degc = jnp.maximum(dga_ref[...] + dgb_ref[...], 1.0)
    d = _silu(agg_ref[...] / degc + bc_ref[...] + rtp_ref[...])
    h = d + gte_ref[...]
    bs, bs2 = _msum(h, i)
    _stats_update(s_ref, p, i, bs, bs2)

    @pl.when(p == 1)
    def _():
        hn = _ln_apply(h, s_ref, float(N * 32), g_ref, b_ref)
        hp_ref[...] = jnp.dot(hn, wa_ref[...],
                              preferred_element_type=jnp.float32)
        rt_ref[...] = jnp.dot(hn, wr_ref[...],
                              preferred_element_type=jnp.float32)
        if emit_d0:
            d0_ref[...] = d


def _layer3_body(agg_ref, rtp_ref, d0_ref, ga_ref, gb_ref,
                 bc_ref, lga_ref, lgb_ref, lba_ref, lbb_ref,
                 waa_ref, wab_ref, wra_ref, wrb_ref,
                 hp_ref, rt_ref, s_ref):
    p = pl.program_id(0)
    i = pl.program_id(1)
    u0 = _silu(agg_ref[...] + bc_ref[...] + rtp_ref[...])
    ha = u0 + ga_ref[...]
    hb = d0_ref[...] + gb_ref[...]
    bsa, bsa2 = _msum(ha, i)
    bsb, bsb2 = _msum(hb, i)
    _stats_update(s_ref, p, i, bsa + bsb, bsa2 + bsb2)

    @pl.when(p == 1)
    def _():
        cnt = float(N * 64)
        hna = _ln_apply(ha, s_ref, cnt, lga_ref, lba_ref)
        hnb = _ln_apply(hb, s_ref, cnt, lgb_ref, lbb_ref)
        hp_ref[...] = (
            jnp.dot(hna, waa_ref[...], preferred_element_type=jnp.float32)
            + jnp.dot(hnb, wab_ref[...], preferred_element_type=jnp.float32))
        rt_ref[...] = (
            jnp.dot(hna, wra_ref[...], preferred_element_type=jnp.float32)
            + jnp.dot(hnb, wrb_ref[...], preferred_element_type=jnp.float32))


def _layer4_body(agg_ref, rtp_ref, bc_ref, wa_ref, wr_ref, hp_ref, rt_ref):
    u1 = _silu(agg_ref[...] + bc_ref[...] + rtp_ref[...])
    hp_ref[...] = jnp.dot(u1, wa_ref[...], preferred_element_type=jnp.float32)
    rt_ref[...] = jnp.dot(u1, wr_ref[...], preferred_element_type=jnp.float32)


def _layer5_body(agg_ref, rtp_ref, bc_ref, out_ref):
    out_ref[...] = agg_ref[...] + bc_ref[...] + rtp_ref[...]


def _call_layer(body, n_in, n_out, two_phase=True, par_shapes=None):
    if two_phase:
        grid = (2, NBLK)
        imap = lambda p, i: (i, 0)
        fmap = lambda p, i: (0, 0)
    else:
        grid = (NBLK,)
        imap = lambda i: (i, 0)
        fmap = lambda i: (0, 0)
    in_specs = [pl.BlockSpec((BR, 128), imap) for _ in range(n_in)]
    in_specs += [pl.BlockSpec(s, fmap) for s in par_shapes]
    out_specs = [pl.BlockSpec((BR, 128), imap) for _ in range(n_out)]
    out_shape = [jax.ShapeDtypeStruct((PR, 128), jnp.float32)
                 for _ in range(n_out)]
    scratch = [pltpu.SMEM((2,), jnp.float32)] if two_phase else []
    return pl.pallas_call(
        body, grid=grid, in_specs=in_specs, out_specs=out_specs,
        out_shape=out_shape, scratch_shapes=scratch)


@functools.cache
def _mesh():
    return plsc.VectorSubcoreMesh(core_axis_name="c", subcore_axis_name="s",
                                  num_cores=2, num_subcores=16)


_CH = 512
_NMB = 200


def _deg_loop(d1_hbm, dacc, ones_v, didx, isd, c, s):
    ebase = (c * 16 + s) * (EP // 32)
    nch = EP // 32 // _CH

    def iload(m, b):
        pltpu.async_copy(d1_hbm.at[pl.ds(ebase + m * _CH, _CH)], didx[b],
                         isd[b])

    def iwait(m, b):
        pltpu.make_async_copy(d1_hbm.at[pl.ds(ebase + m * _CH, _CH)],
                              didx[b], isd[b]).wait()

    iload(0, 0)

    @pl.loop(0, nch // 2)
    def _pair(mm):
        m = mm * 2
        iload(m + 1, 1)
        iwait(m, 0)
        pltpu.sync_copy(ones_v, dacc.at[didx[0]], add=True)

        @pl.when(m + 2 < nch)
        def _():
            iload(m + 2, 0)

        iwait(m + 1, 1)
        pltpu.sync_copy(ones_v, dacc.at[didx[1]], add=True)


def _sc_prelude_kernel(d1_hbm, t1_hbm, tb0, tb1, tb2, tb3a, tb3b,
                       ones_hbm, zeros_hbm,
                       deg_hbm, g0_hbm, g1_hbm, g2_hbm, g3a_hbm, g3b_hbm,
                       dacc, zbuf, ones_v, didx0, didx1, tidx, gv,
                       isd0, isd1, sem):
    c = lax.axis_index("c")
    s = lax.axis_index("s")
    w = c * 16 + s
    pltpu.sync_copy(zeros_hbm, zbuf)
    base = s * 6400

    @pl.loop(0, 16)
    def _zero(k):
        pltpu.sync_copy(zbuf, dacc.at[pl.ds(base + k * 400, 400)])

    plsc.subcore_barrier()
    pltpu.sync_copy(ones_hbm, ones_v)
    _deg_loop(d1_hbm, dacc, ones_v, (didx0, didx1), (isd0, isd1), c, s)
    plsc.subcore_barrier()

    @pl.when(c == 0)
    def _():
        pltpu.sync_copy(dacc.at[pl.ds(base, 6400)],
                        deg_hbm.at[pl.ds(base, 6400), pl.ds(0, 16)])

    @pl.when(c == 1)
    def _():
        pltpu.sync_copy(dacc.at[pl.ds(base, 6400)],
                        deg_hbm.at[pl.ds(base, 6400), pl.ds(16, 16)])

    pltpu.sync_copy(t1_hbm.at[pl.ds(w * 3200, 3200)], tidx)

    @pl.loop(0, 20)
    def _te(j):
        nbase = w * 3200 + j * 160
        for tb, g in ((tb0, g0_hbm), (tb1, g1_hbm), (tb2, g2_hbm),
                      (tb3a, g3a_hbm), (tb3b, g3b_hbm)):
            pltpu.async_copy(tb.at[tidx.at[pl.ds(j * 160, 160)]], gv,
                             sem).wait()
            pltpu.sync_copy(gv, g.at[pl.ds(nbase, 160)])


@functools.cache
def _sc_prelude():
    return pl.kernel(
        _sc_prelude_kernel, mesh=_mesh(),
        out_type=(
            jax.ShapeDtypeStruct((NA, 32), jnp.float32),
            jax.ShapeDtypeStruct((NA, 32), jnp.float32),
            jax.ShapeDtypeStruct((NA, 32), jnp.float32),
            jax.ShapeDtypeStruct((NA, 32), jnp.float32),
            jax.ShapeDtypeStruct((NA, 32), jnp.float32),
            jax.ShapeDtypeStruct((NA, 32), jnp.float32),
        ),
        compiler_params=pltpu.CompilerParams(use_tc_tiling_on_sc=False),
        scratch_types=[
            pltpu.VMEM_SHARED((NA, 16), jnp.float32),
            pltpu.VMEM((400, 16), jnp.float32),
            pltpu.VMEM((_CH, 16), jnp.float32),
            pltpu.VMEM((_CH,), jnp.int32),
            pltpu.VMEM((_CH,), jnp.int32),
            pltpu.VMEM((3200,), jnp.int32),
            pltpu.VMEM((160, 32), jnp.float32),
            pltpu.SemaphoreType.DMA,
            pltpu.SemaphoreType.DMA,
            pltpu.SemaphoreType.DMA,
        ],
    )


def _conv_edge_loop(hp_hbm, s1_hbm, d1_hbm, acc, sidx, didx, rv,
                    gs, iss, isd, s):
    ebase = s * (_NMB * _CH)

    def iload(m, b):
        pltpu.async_copy(s1_hbm.at[pl.ds(ebase + m * _CH, _CH)], sidx[b],
                         iss[b])
        pltpu.async_copy(d1_hbm.at[pl.ds(ebase + m * _CH, _CH)], didx[b],
                         isd[b])

    def iwait_s(m, b):
        pltpu.make_async_copy(s1_hbm.at[pl.ds(ebase + m * _CH, _CH)],
                              sidx[b], iss[b]).wait()

    def iwait_d(m, b):
        pltpu.make_async_copy(d1_hbm.at[pl.ds(ebase + m * _CH, _CH)],
                              didx[b], isd[b]).wait()

    def gfire(b):
        pltpu.async_copy(hp_hbm.at[sidx[b]], rv[b], gs[b])

    def gwait(b):
        pltpu.make_async_copy(hp_hbm.at[sidx[b]], rv[b], gs[b]).wait()

    def scat(b):
        pltpu.sync_copy(rv[b], acc.at[didx[b]], add=True)

    iload(0, 0)
    iwait_s(0, 0)
    gfire(0)
    iload(1, 1)

    @pl.loop(0, _NMB // 2)
    def _pair(mm):
        m = mm * 2
        iwait_s(m + 1, 1)
        gfire(1)
        gwait(0)

        @pl.when(m + 2 < _NMB)
        def _():
            iload(m + 2, 0)

        iwait_d(m, 0)
        scat(0)

        @pl.when(m + 2 < _NMB)
        def _():
            iwait_s(m + 2, 0)
            gfire(0)

        gwait(1)

        @pl.when(m + 3 < _NMB)
        def _():
            iload(m + 3, 1)

        iwait_d(m + 1, 1)
        scat(1)


def _sc_conv_kernel(hpa_hbm, hpb_hbm, s1_hbm, d1_hbm, zeros_hbm, agg_hbm,
                    acc, zbuf, sidx0, didx0, sidx1, didx1, rv0, rv1,
                    gs0, gs1, iss0, iss1, isd0, isd1):
    c = lax.axis_index("c")
    s = lax.axis_index("s")
    pltpu.sync_copy(zeros_hbm, zbuf)
    base = s * 6400

    @pl.loop(0, 16)
    def _zero(k):
        pltpu.sync_copy(zbuf, acc.at[pl.ds(base + k * 400, 400)])

    plsc.subcore_barrier()

    @pl.when(c == 0)
    def _():
        _conv_edge_loop(hpa_hbm, s1_hbm, d1_hbm, acc, (sidx0, sidx1),
                        (didx0, didx1), (rv0, rv1), (gs0, gs1),
                        (iss0, iss1), (isd0, isd1), s)

    @pl.when(c == 1)
    def _():
        _conv_edge_loop(hpb_hbm, s1_hbm, d1_hbm, acc, (sidx0, sidx1),
                        (didx0, didx1), (rv0, rv1), (gs0, gs1),
                        (iss0, iss1), (isd0, isd1), s)

    plsc.subcore_barrier()

    @pl.when(c == 0)
    def _():
        pltpu.sync_copy(acc.at[pl.ds(base, 6400)],
                        agg_hbm.at[pl.ds(base, 6400), pl.ds(0, 16)])

    @pl.when(c == 1)
    def _():
        pltpu.sync_copy(acc.at[pl.ds(base, 6400)],
                        agg_hbm.at[pl.ds(base, 6400), pl.ds(16, 16)])


@functools.cache
def _sc_conv():
    return pl.kernel(
        _sc_conv_kernel, mesh=_mesh(),
        out_type=jax.ShapeDtypeStruct((NA, 32), jnp.float32),
        compiler_params=pltpu.CompilerParams(use_tc_tiling_on_sc=False),
        scratch_types=[
            pltpu.VMEM_SHARED((NA, 16), jnp.float32),
            pltpu.VMEM((400, 16), jnp.float32),
            pltpu.VMEM((_CH,), jnp.int32),
            pltpu.VMEM((_CH,), jnp.int32),
            pltpu.VMEM((_CH,), jnp.int32),
            pltpu.VMEM((_CH,), jnp.int32),
            pltpu.VMEM((_CH, 16), jnp.float32),
            pltpu.VMEM((_CH, 16), jnp.float32),
            pltpu.SemaphoreType.DMA,
            pltpu.SemaphoreType.DMA,
            pltpu.SemaphoreType.DMA,
            pltpu.SemaphoreType.DMA,
            pltpu.SemaphoreType.DMA,
            pltpu.SemaphoreType.DMA,
        ],
    )


def _sin_table():
    t = np.arange(1000, dtype=np.float32).reshape(1000, 1)
    wk = (1.0 / 10000 ** (2.0 * np.arange(100, dtype=np.float32) / 100)
          ).reshape(1, 100)
    emb = np.zeros((1000, 100), dtype=np.float32)
    emb[:, 0::2] = np.sin(t * wk[:, 0::2])
    emb[:, 1::2] = np.cos(t * wk[:, 0::2])
    out = np.zeros((1024, 128), dtype=np.float32)
    out[:1000, :100] = emb
    return out


_EMBP = _sin_table()
_ZEROS = np.zeros((400, 16), dtype=np.float32)
_ONES = np.ones((_CH, 16), dtype=np.float32)
_EYE4 = np.eye(4, dtype=np.float32)


def _padw(w):
    out = jnp.zeros((128, w.shape[0]), jnp.float32)
    return out.at[:100, :].set(w.T)


def _bd(w):
    return jnp.kron(jnp.asarray(_EYE4), w)


def kernel(x, edge_index, t, params):
    p = params
    src = edge_index[0]
    dst = edge_index[1]
    pad = EP - E
    s_pad = jnp.concatenate([src, jnp.zeros((pad,), jnp.int32)])
    d_pad = jnp.concatenate([dst, jnp.full((pad,), TRASH, jnp.int32)])
    t_pad = jnp.concatenate([t, jnp.zeros((NA - N,), jnp.int32)])
    embp = jnp.asarray(_EMBP)
    zeros = jnp.asarray(_ZEROS)
    ones = jnp.asarray(_ONES)

    wts = (_padw(p['te_d0_W']), _padw(p['te_d1_W']), _padw(p['te_u0_W']),
           _padw(p['te_u1_W'])[:, :32], _padw(p['te_u1_W'])[:, 32:])
    bs = (p['te_d0_b'].reshape(1, 32), p['te_d1_b'].reshape(1, 32),
          p['te_u0_b'].reshape(1, 32), p['te_u1_b'][:32].reshape(1, 32),
          p['te_u1_b'][32:].reshape(1, 32))
    tb0, tb1, tb2, tb3a, tb3b = _make_tables(embp, wts, bs)

    deg, g0, g1, g2, g3a, g3b = _sc_prelude()(
        d_pad, t_pad, tb0, tb1, tb2, tb3a, tb3b, ones, zeros)

    r128 = lambda v: jnp.tile(v.reshape(1, 32), (1, 4))
    pk = lambda v: v.reshape(PR, 128)
    un = lambda v: v.reshape(NA, 32)

    deg_sw = jnp.concatenate([deg[:, 16:], deg[:, :16]], axis=1)
    dga, dgb = pk(deg), pk(deg_sw)

    xp = pk(jnp.pad(x, ((0, NA - N), (0, 0))))
    g0p, g1p, g2p, g3ap, g3bp = pk(g0), pk(g1), pk(g2), pk(g3a), pk(g3b)

    l0 = _call_layer(_layer0_body, 2, 2,
                     par_shapes=[(1, 128), (1, 128), (128, 128), (128, 128)])
    halves = lambda v: (v.reshape(NA, 32)[:, :16], v.reshape(NA, 32)[:, 16:])
    hp, rt = l0(xp, g0p, r128(p['ln_d0_g']), r128(p['ln_d0_b']),
                _bd(p['sage0_Wl'].T), _bd(p['sage0_Wr'].T))
    agg = _sc_conv()(*halves(hp), s_pad, d_pad, zeros)

    l1 = _call_layer(functools.partial(_mid_body, True), 5, 3,
                     par_shapes=[(1, 128)] * 3 + [(128, 128)] * 2)
    hp, rt, d0 = l1(pk(agg), dga, dgb, rt, g1p,
                    r128(p['sage0_bl']), r128(p['ln_d1_g']),
                    r128(p['ln_d1_b']), _bd(p['sage1_Wl'].T),
                    _bd(p['sage1_Wr'].T))
    agg = _sc_conv()(*halves(hp), s_pad, d_pad, zeros)

    l2 = _call_layer(functools.partial(_mid_body, False), 5, 2,
                     par_shapes=[(1, 128)] * 3 + [(128, 128)] * 2)
    hp, rt = l2(pk(agg), dga, dgb, rt, g2p,
                r128(p['sage1_bl']), r128(p['ln_u0_g']),
                r128(p['ln_u0_b']), _bd(p['up0_Wrel'].T),
                _bd(p['up0_Wroot'].T))
    agg = _sc_conv()(*halves(hp), s_pad, d_pad, zeros)

    wrel = p['up1_Wrel'].T
    wroot = p['up1_Wroot'].T
    l3 = _call_layer(_layer3_body, 5, 2,
                     par_shapes=[(1, 128)] * 5 + [(128, 128)] * 4)
    hp, rt = l3(pk(agg), rt, d0, g3ap, g3bp,
                r128(p['up0_brel']),
                r128(p['ln_u1_g'][:32]), r128(p['ln_u1_g'][32:]),
                r128(p['ln_u1_b'][:32]), r128(p['ln_u1_b'][32:]),
                _bd(wrel[:32]), _bd(wrel[32:]),
                _bd(wroot[:32]), _bd(wroot[32:]))
    agg = _sc_conv()(*halves(hp), s_pad, d_pad, zeros)

    l4 = _call_layer(_layer4_body, 2, 2, two_phase=False,
                     par_shapes=[(1, 128), (128, 128), (128, 128)])
    hp, rt = l4(pk(agg), rt, r128(p['up1_brel']),
                _bd(p['last_Wrel'].T), _bd(p['last_Wroot'].T))
    agg = _sc_conv()(*halves(hp), s_pad, d_pad, zeros)

    l5 = _call_layer(_layer5_body, 2, 1, two_phase=False,
                     par_shapes=[(1, 128)])
    (out,) = l5(pk(agg), rt, r128(p['last_brel']))
    return un(out)[:N]


# speedup vs baseline: 1.0382x; 1.0382x over previous
import functools
import numpy as np
import jax
import jax.numpy as jnp
from jax import lax
from jax.experimental import pallas as pl
from jax.experimental.pallas import tpu as pltpu
from jax.experimental.pallas import tpu_sc as plsc

N = 100000
E = 1600000
EP = 1638400
NA = 102400
PR = NA * 32 // 128
TRASH = N
BR = 256
NBLK = PR // BR
EPS = 1e-5


def _silu(v):
    return v * (1.0 / (1.0 + jnp.exp(-v)))


def _msum(h, i):
    rid = lax.broadcasted_iota(jnp.int32, h.shape, 0)
    lid = lax.broadcasted_iota(jnp.int32, h.shape, 1)
    node = (i * BR + rid) * 4 + lid // 32
    hm = h * (node < N).astype(jnp.float32)
    return jnp.sum(hm), jnp.sum(hm * h)


def _tables_body(emb_ref, w0, w1, w2, w3a, w3b, b0, b1, b2, b3a, b3b,
                 t0, t1, t2, t3a, t3b):
    e = emb_ref[...]
    for w, b, o in ((w0, b0, t0), (w1, b1, t1), (w2, b2, t2),
                    (w3a, b3a, t3a), (w3b, b3b, t3b)):
        o[...] = _silu(jnp.dot(e, w[...], preferred_element_type=jnp.float32)
                       + b[...])


def _make_tables(embp, wts, bs):
    out = tuple(jax.ShapeDtypeStruct((1024, 32), jnp.float32) for _ in range(5))
    return pl.pallas_call(_tables_body, out_shape=out)(embp, *wts, *bs)


def _stats_update(s_ref, p, i, bs, bs2):
    @pl.when(jnp.logical_and(p == 0, i == 0))
    def _():
        s_ref[0] = bs
        s_ref[1] = bs2

    @pl.when(jnp.logical_and(p == 0, i > 0))
    def _():
        s_ref[0] += bs
        s_ref[1] += bs2


def _ln_apply(h, s_ref, count, g_ref, b_ref):
    mu = s_ref[0] / count
    var = s_ref[1] / count - mu * mu
    rs = lax.rsqrt(var + EPS)
    return (h - mu) * rs * g_ref[...] + b_ref[...]


def _layer0_body(x_ref, g0_ref, g_ref, b_ref, wa_ref, wr_ref,
                 hp_ref, rt_ref, s_ref):
    p = pl.program_id(0)
    i = pl.program_id(1)
    h = x_ref[...] + g0_ref[...]
    bs, bs2 = _msum(h, i)
    _stats_update(s_ref, p, i, bs, bs2)

    @pl.when(p == 1)
    def _():
        hn = _ln_apply(h, s_ref, float(N * 32), g_ref, b_ref)
        hp_ref[...] = jnp.dot(hn, wa_ref[...],
                              preferred_element_type=jnp.float32)
        rt_ref[...] = jnp.dot(hn, wr_ref[...],
                              preferred_element_type=jnp.float32)


def _mid_body(emit_d0, agg_ref, dga_ref, dgb_ref, rtp_ref, gte_ref,
              bc_ref, g_ref, b_ref, wa_ref, wr_ref, *rest):
    if emit_d0:
        hp_ref, rt_ref, d0_ref, s_ref = rest
    else:
        hp_ref, rt_ref, s_ref = rest
    p = pl.program_id(0)
    i = pl.program_id(1)
    degc = jnp.maximum(dga_ref[...] + dgb_ref[...], 1.0)
    d = _silu(agg_ref[...] / degc + bc_ref[...] + rtp_ref[...])
    h = d + gte_ref[...]
    bs, bs2 = _msum(h, i)
    _stats_update(s_ref, p, i, bs, bs2)

    @pl.when(p == 1)
    def _():
        hn = _ln_apply(h, s_ref, float(N * 32), g_ref, b_ref)
        hp_ref[...] = jnp.dot(hn, wa_ref[...],
                              preferred_element_type=jnp.float32)
        rt_ref[...] = jnp.dot(hn, wr_ref[...],
                              preferred_element_type=jnp.float32)
        if emit_d0:
            d0_ref[...] = d


def _layer3_body(agg_ref, rtp_ref, d0_ref, ga_ref, gb_ref,
                 bc_ref, lga_ref, lgb_ref, lba_ref, lbb_ref,
                 waa_ref, wab_ref, wra_ref, wrb_ref,
                 hp_ref, rt_ref, s_ref):
    p = pl.program_id(0)
    i = pl.program_id(1)
    u0 = _silu(agg_ref[...] + bc_ref[...] + rtp_ref[...])
    ha = u0 + ga_ref[...]
    hb = d0_ref[...] + gb_ref[...]
    bsa, bsa2 = _msum(ha, i)
    bsb, bsb2 = _msum(hb, i)
    _stats_update(s_ref, p, i, bsa + bsb, bsa2 + bsb2)

    @pl.when(p == 1)
    def _():
        cnt = float(N * 64)
        hna = _ln_apply(ha, s_ref, cnt, lga_ref, lba_ref)
        hnb = _ln_apply(hb, s_ref, cnt, lgb_ref, lbb_ref)
        hp_ref[...] = (
            jnp.dot(hna, waa_ref[...], preferred_element_type=jnp.float32)
            + jnp.dot(hnb, wab_ref[...], preferred_element_type=jnp.float32))
        rt_ref[...] = (
            jnp.dot(hna, wra_ref[...], preferred_element_type=jnp.float32)
            + jnp.dot(hnb, wrb_ref[...], preferred_element_type=jnp.float32))


def _layer4_body(agg_ref, rtp_ref, bc_ref, wa_ref, wr_ref, hp_ref, rt_ref):
    u1 = _silu(agg_ref[...] + bc_ref[...] + rtp_ref[...])
    hp_ref[...] = jnp.dot(u1, wa_ref[...], preferred_element_type=jnp.float32)
    rt_ref[...] = jnp.dot(u1, wr_ref[...], preferred_element_type=jnp.float32)


def _layer5_body(agg_ref, rtp_ref, bc_ref, out_ref):
    out_ref[...] = agg_ref[...] + bc_ref[...] + rtp_ref[...]


def _call_layer(body, n_in, n_out, two_phase=True, par_shapes=None):
    if two_phase:
        grid = (2, NBLK)
        imap = lambda p, i: (i, 0)
        fmap = lambda p, i: (0, 0)
    else:
        grid = (NBLK,)
        imap = lambda i: (i, 0)
        fmap = lambda i: (0, 0)
    in_specs = [pl.BlockSpec((BR, 128), imap) for _ in range(n_in)]
    in_specs += [pl.BlockSpec(s, fmap) for s in par_shapes]
    out_specs = [pl.BlockSpec((BR, 128), imap) for _ in range(n_out)]
    out_shape = [jax.ShapeDtypeStruct((PR, 128), jnp.float32)
                 for _ in range(n_out)]
    scratch = [pltpu.SMEM((2,), jnp.float32)] if two_phase else []
    return pl.pallas_call(
        body, grid=grid, in_specs=in_specs, out_specs=out_specs,
        out_shape=out_shape, scratch_shapes=scratch)


@functools.cache
def _mesh():
    return plsc.VectorSubcoreMesh(core_axis_name="c", subcore_axis_name="s",
                                  num_cores=2, num_subcores=16)


_CH = 512
_NMB = 200


def _deg_loop(d1_hbm, dacc, ones_v, didx, isd, c, s):
    ebase = (c * 16 + s) * (EP // 32)
    nch = EP // 32 // _CH

    def iload(m, b):
        pltpu.async_copy(d1_hbm.at[pl.ds(ebase + m * _CH, _CH)], didx[b],
                         isd[b])

    def iwait(m, b):
        pltpu.make_async_copy(d1_hbm.at[pl.ds(ebase + m * _CH, _CH)],
                              didx[b], isd[b]).wait()

    iload(0, 0)

    @pl.loop(0, nch // 2)
    def _pair(mm):
        m = mm * 2
        iload(m + 1, 1)
        iwait(m, 0)
        pltpu.sync_copy(ones_v, dacc.at[didx[0]], add=True)

        @pl.when(m + 2 < nch)
        def _():
            iload(m + 2, 0)

        iwait(m + 1, 1)
        pltpu.sync_copy(ones_v, dacc.at[didx[1]], add=True)


def _sc_prelude_kernel(d1_hbm, t1_hbm, tb0, tb1, tb2, tb3a, tb3b,
                       ones_hbm, zeros_hbm,
                       deg_hbm, g0_hbm, g1_hbm, g2_hbm, g3a_hbm, g3b_hbm,
                       dacc, zbuf, ones_v, didx0, didx1, tidx, gv,
                       isd0, isd1, sem):
    c = lax.axis_index("c")
    s = lax.axis_index("s")
    w = c * 16 + s
    pltpu.sync_copy(zeros_hbm, zbuf)
    base = s * 6400

    @pl.loop(0, 16)
    def _zero(k):
        pltpu.sync_copy(zbuf, dacc.at[pl.ds(base + k * 400, 400)])

    plsc.subcore_barrier()
    pltpu.sync_copy(ones_hbm, ones_v)
    _deg_loop(d1_hbm, dacc, ones_v, (didx0, didx1), (isd0, isd1), c, s)
    plsc.subcore_barrier()

    @pl.when(c == 0)
    def _():
        pltpu.sync_copy(dacc.at[pl.ds(base, 6400)],
                        deg_hbm.at[pl.ds(base, 6400), pl.ds(0, 16)])

    @pl.when(c == 1)
    def _():
        pltpu.sync_copy(dacc.at[pl.ds(base, 6400)],
                        deg_hbm.at[pl.ds(base, 6400), pl.ds(16, 16)])

    pltpu.sync_copy(t1_hbm.at[pl.ds(w * 3200, 3200)], tidx)

    @pl.loop(0, 20)
    def _te(j):
        nbase = w * 3200 + j * 160
        for tb, g in ((tb0, g0_hbm), (tb1, g1_hbm), (tb2, g2_hbm),
                      (tb3a, g3a_hbm), (tb3b, g3b_hbm)):
            pltpu.async_copy(tb.at[tidx.at[pl.ds(j * 160, 160)]], gv,
                             sem).wait()
            pltpu.sync_copy(gv, g.at[pl.ds(nbase, 160)])


@functools.cache
def _sc_prelude():
    return pl.kernel(
        _sc_prelude_kernel, mesh=_mesh(),
        out_type=(
            jax.ShapeDtypeStruct((NA, 32), jnp.float32),
            jax.ShapeDtypeStruct((NA, 32), jnp.float32),
            jax.ShapeDtypeStruct((NA, 32), jnp.float32),
            jax.ShapeDtypeStruct((NA, 32), jnp.float32),
            jax.ShapeDtypeStruct((NA, 32), jnp.float32),
            jax.ShapeDtypeStruct((NA, 32), jnp.float32),
        ),
        compiler_params=pltpu.CompilerParams(use_tc_tiling_on_sc=False),
        scratch_types=[
            pltpu.VMEM_SHARED((NA, 16), jnp.float32),
            pltpu.VMEM((400, 16), jnp.float32),
            pltpu.VMEM((_CH, 16), jnp.float32),
            pltpu.VMEM((_CH,), jnp.int32),
            pltpu.VMEM((_CH,), jnp.int32),
            pltpu.VMEM((3200,), jnp.int32),
            pltpu.VMEM((160, 32), jnp.float32),
            pltpu.SemaphoreType.DMA,
            pltpu.SemaphoreType.DMA,
            pltpu.SemaphoreType.DMA,
        ],
    )


def _conv_edge_loop(hp_hbm, s1_hbm, d1_hbm, acc, sidx, didx, rv,
                    gs, iss, isd, s):
    ebase = s * (_NMB * _CH)

    def iload(m, b):
        pltpu.async_copy(s1_hbm.at[pl.ds(ebase + m * _CH, _CH)], sidx[b],
                         iss[b])
        pltpu.async_copy(d1_hbm.at[pl.ds(ebase + m * _CH, _CH)], didx[b],
                         isd[b])

    def iwait_s(m, b):
        pltpu.make_async_copy(s1_hbm.at[pl.ds(ebase + m * _CH, _CH)],
                              sidx[b], iss[b]).wait()

    def iwait_d(m, b):
        pltpu.make_async_copy(d1_hbm.at[pl.ds(ebase + m * _CH, _CH)],
                              didx[b], isd[b]).wait()

    def gfire(b):
        pltpu.async_copy(hp_hbm.at[sidx[b]], rv[b], gs[b])

    def gwait(b):
        pltpu.make_async_copy(hp_hbm.at[sidx[b]], rv[b], gs[b]).wait()

    def scat(b):
        pltpu.sync_copy(rv[b], acc.at[didx[b]], add=True)

    iload(0, 0)
    iwait_s(0, 0)
    gfire(0)
    iload(1, 1)

    @pl.loop(0, _NMB // 2)
    def _pair(mm):
        m = mm * 2
        iwait_s(m + 1, 1)
        gfire(1)
        gwait(0)

        @pl.when(m + 2 < _NMB)
        def _():
            iload(m + 2, 0)

        iwait_d(m, 0)
        scat(0)

        @pl.when(m + 2 < _NMB)
        def _():
            iwait_s(m + 2, 0)
            gfire(0)

        gwait(1)

        @pl.when(m + 3 < _NMB)
        def _():
            iload(m + 3, 1)

        iwait_d(m + 1, 1)
        scat(1)


def _sc_conv_kernel(hp_hbm, s2a_hbm, s2b_hbm, d1_hbm, zeros_hbm, agg_hbm,
                    acc, zbuf, sidx0, didx0, sidx1, didx1, rv0, rv1,
                    gs0, gs1, iss0, iss1, isd0, isd1):
    c = lax.axis_index("c")
    s = lax.axis_index("s")
    pltpu.sync_copy(zeros_hbm, zbuf)
    base = s * 6400

    @pl.loop(0, 16)
    def _zero(k):
        pltpu.sync_copy(zbuf, acc.at[pl.ds(base + k * 400, 400)])

    plsc.subcore_barrier()

    @pl.when(c == 0)
    def _():
        _conv_edge_loop(hp_hbm, s2a_hbm, d1_hbm, acc, (sidx0, sidx1),
                        (didx0, didx1), (rv0, rv1), (gs0, gs1),
                        (iss0, iss1), (isd0, isd1), s)

    @pl.when(c == 1)
    def _():
        _conv_edge_loop(hp_hbm, s2b_hbm, d1_hbm, acc, (sidx0, sidx1),
                        (didx0, didx1), (rv0, rv1), (gs0, gs1),
                        (iss0, iss1), (isd0, isd1), s)

    plsc.subcore_barrier()

    @pl.when(c == 0)
    def _():
        pltpu.sync_copy(acc.at[pl.ds(base, 6400)],
                        agg_hbm.at[pl.ds(base, 6400), pl.ds(0, 16)])

    @pl.when(c == 1)
    def _():
        pltpu.sync_copy(acc.at[pl.ds(base, 6400)],
                        agg_hbm.at[pl.ds(base, 6400), pl.ds(16, 16)])


@functools.cache
def _sc_conv():
    return pl.kernel(
        _sc_conv_kernel, mesh=_mesh(),
        out_type=jax.ShapeDtypeStruct((NA, 32), jnp.float32),
        compiler_params=pltpu.CompilerParams(use_tc_tiling_on_sc=False),
        scratch_types=[
            pltpu.VMEM_SHARED((NA, 16), jnp.float32),
            pltpu.VMEM((400, 16), jnp.float32),
            pltpu.VMEM((_CH,), jnp.int32),
            pltpu.VMEM((_CH,), jnp.int32),
            pltpu.VMEM((_CH,), jnp.int32),
            pltpu.VMEM((_CH,), jnp.int32),
            pltpu.VMEM((_CH, 16), jnp.float32),
            pltpu.VMEM((_CH, 16), jnp.float32),
            pltpu.SemaphoreType.DMA,
            pltpu.SemaphoreType.DMA,
            pltpu.SemaphoreType.DMA,
            pltpu.SemaphoreType.DMA,
            pltpu.SemaphoreType.DMA,
            pltpu.SemaphoreType.DMA,
        ],
    )


def _sin_table():
    t = np.arange(1000, dtype=np.float32).reshape(1000, 1)
    wk = (1.0 / 10000 ** (2.0 * np.arange(100, dtype=np.float32) / 100)
          ).reshape(1, 100)
    emb = np.zeros((1000, 100), dtype=np.float32)
    emb[:, 0::2] = np.sin(t * wk[:, 0::2])
    emb[:, 1::2] = np.cos(t * wk[:, 0::2])
    out = np.zeros((1024, 128), dtype=np.float32)
    out[:1000, :100] = emb
    return out


_EMBP = _sin_table()
_ZEROS = np.zeros((400, 16), dtype=np.float32)
_ONES = np.ones((_CH, 16), dtype=np.float32)
_EYE4 = np.eye(4, dtype=np.float32)


def _padw(w):
    out = jnp.zeros((128, w.shape[0]), jnp.float32)
    return out.at[:100, :].set(w.T)


def _bd(w):
    return jnp.kron(jnp.asarray(_EYE4), w)


def kernel(x, edge_index, t, params):
    p = params
    src = edge_index[0]
    dst = edge_index[1]
    pad = EP - E
    s_pad = jnp.concatenate([src, jnp.zeros((pad,), jnp.int32)])
    d_pad = jnp.concatenate([dst, jnp.full((pad,), TRASH, jnp.int32)])
    t_pad = jnp.concatenate([t, jnp.zeros((NA - N,), jnp.int32)])
    embp = jnp.asarray(_EMBP)
    zeros = jnp.asarray(_ZEROS)
    ones = jnp.asarray(_ONES)

    wts = (_padw(p['te_d0_W']), _padw(p['te_d1_W']), _padw(p['te_u0_W']),
           _padw(p['te_u1_W'])[:, :32], _padw(p['te_u1_W'])[:, 32:])
    bs = (p['te_d0_b'].reshape(1, 32), p['te_d1_b'].reshape(1, 32),
          p['te_u0_b'].reshape(1, 32), p['te_u1_b'][:32].reshape(1, 32),
          p['te_u1_b'][32:].reshape(1, 32))
    tb0, tb1, tb2, tb3a, tb3b = _make_tables(embp, wts, bs)

    deg, g0, g1, g2, g3a, g3b = _sc_prelude()(
        d_pad, t_pad, tb0, tb1, tb2, tb3a, tb3b, ones, zeros)

    r128 = lambda v: jnp.tile(v.reshape(1, 32), (1, 4))
    pk = lambda v: v.reshape(PR, 128)
    un = lambda v: v.reshape(NA, 32)

    deg_sw = jnp.concatenate([deg[:, 16:], deg[:, :16]], axis=1)
    dga, dgb = pk(deg), pk(deg_sw)

    xp = pk(jnp.pad(x, ((0, NA - N), (0, 0))))
    g0p, g1p, g2p, g3ap, g3bp = pk(g0), pk(g1), pk(g2), pk(g3a), pk(g3b)

    l0 = _call_layer(_layer0_body, 2, 2,
                     par_shapes=[(1, 128), (1, 128), (128, 128), (128, 128)])
    s2a = s_pad * 2
    s2b = s_pad * 2 + 1
    half = lambda v: v.reshape(NA * 2, 16)
    hp, rt = l0(xp, g0p, r128(p['ln_d0_g']), r128(p['ln_d0_b']),
                _bd(p['sage0_Wl'].T), _bd(p['sage0_Wr'].T))
    agg = _sc_conv()(half(hp), s2a, s2b, d_pad, zeros)

    l1 = _call_layer(functools.partial(_mid_body, True), 5, 3,
                     par_shapes=[(1, 128)] * 3 + [(128, 128)] * 2)
    hp, rt, d0 = l1(pk(agg), dga, dgb, rt, g1p,
                    r128(p['sage0_bl']), r128(p['ln_d1_g']),
                    r128(p['ln_d1_b']), _bd(p['sage1_Wl'].T),
                    _bd(p['sage1_Wr'].T))
    agg = _sc_conv()(half(hp), s2a, s2b, d_pad, zeros)

    l2 = _call_layer(functools.partial(_mid_body, False), 5, 2,
                     par_shapes=[(1, 128)] * 3 + [(128, 128)] * 2)
    hp, rt = l2(pk(agg), dga, dgb, rt, g2p,
                r128(p['sage1_bl']), r128(p['ln_u0_g']),
                r128(p['ln_u0_b']), _bd(p['up0_Wrel'].T),
                _bd(p['up0_Wroot'].T))
    agg = _sc_conv()(half(hp), s2a, s2b, d_pad, zeros)

    wrel = p['up1_Wrel'].T
    wroot = p['up1_Wroot'].T
    l3 = _call_layer(_layer3_body, 5, 2,
                     par_shapes=[(1, 128)] * 5 + [(128, 128)] * 4)
    hp, rt = l3(pk(agg), rt, d0, g3ap, g3bp,
                r128(p['up0_brel']),
                r128(p['ln_u1_g'][:32]), r128(p['ln_u1_g'][32:]),
                r128(p['ln_u1_b'][:32]), r128(p['ln_u1_b'][32:]),
                _bd(wrel[:32]), _bd(wrel[32:]),
                _bd(wroot[:32]), _bd(wroot[32:]))
    agg = _sc_conv()(half(hp), s2a, s2b, d_pad, zeros)

    l4 = _call_layer(_layer4_body, 2, 2, two_phase=False,
                     par_shapes=[(1, 128), (128, 128), (128, 128)])
    hp, rt = l4(pk(agg), rt, r128(p['up1_brel']),
                _bd(p['last_Wrel'].T), _bd(p['last_Wroot'].T))
    agg = _sc_conv()(half(hp), s2a, s2b, d_pad, zeros)

    l5 = _call_layer(_layer5_body, 2, 1, two_phase=False,
                     par_shapes=[(1, 128)])
    (out,) = l5(pk(agg), rt, r128(p['last_brel']))
    return un(out)[:N]
